# initial kernel scaffold (unmeasured)
import jax
import jax.numpy as jnp
from jax import lax
from jax.experimental import pallas as pl
from jax.experimental.pallas import tpu as pltpu

N_DEV = 4
M_PER = 1024
K_PER = 1024
N = 8192
BLK_N = 2048
N_BLKS = N // BLK_N
N_HOPS = N_DEV - 1
TOTAL_HOPS = N_BLKS * N_HOPS


def kernel(x, w_mat, scale_x, scale_w):
    x8 = x.astype(jnp.float8_e5m2)
    w8 = w_mat.astype(jnp.float8_e5m2)

    def body(x_ref, w_ref, sx_ref, sw_ref, out_ref,
             comm_ref, send_sems, recv_sems, credit_sem):
        t = pl.program_id(0)
        my = lax.axis_index("i")
        left = lax.rem(my + N_DEV - 1, N_DEV)
        right = lax.rem(my + 1, N_DEV)

        @pl.when(t == 0)
        def _entry_barrier():
            barrier_sem = pltpu.get_barrier_semaphore()
            for nbr in (left, right):
                pl.semaphore_signal(
                    barrier_sem, inc=1,
                    device_id=(nbr,), device_id_type=pl.DeviceIdType.MESH,
                )
            pl.semaphore_wait(barrier_sem, 2)

        def partial_for(chunk):
            xc = x_ref[pl.ds(chunk * M_PER, M_PER), :]
            return lax.dot_general(
                xc, w_ref[:, :], (((1,), (0,)), ((), ())),
                preferred_element_type=jnp.float32,
            )

        send_slot0 = lax.rem(t, 2)
        c0 = lax.rem(my + N_DEV - 1, N_DEV)
        comm_ref[send_slot0] = partial_for(c0).astype(jnp.bfloat16)

        for h in range(N_HOPS):
            send_slot = lax.rem(t + h, 2)
            recv_slot = lax.rem(t + h + 1, 2)

            if h > 0:
                pl.semaphore_wait(credit_sem, 1)
            else:
                @pl.when(t > 0)
                def _():
                    pl.semaphore_wait(credit_sem, 1)

            rdma = pltpu.make_async_remote_copy(
                src_ref=comm_ref.at[send_slot],
                dst_ref=comm_ref.at[recv_slot],
                send_sem=send_sems.at[send_slot],
                recv_sem=recv_sems.at[recv_slot],
                device_id=(right,),
                device_id_type=pl.DeviceIdType.MESH,
            )
            rdma.start()

            c = lax.rem(my + 2 * N_DEV - 2 - h, N_DEV)
            part = partial_for(c)

            rdma.wait()

            if h < N_HOPS - 1:
                comm_ref[recv_slot] = (
                    comm_ref[recv_slot].astype(jnp.float32) + part
                ).astype(jnp.bfloat16)
            else:
                acc = comm_ref[recv_slot].astype(jnp.float32) + part
                y = acc * (sx_ref[0] * sw_ref[0])
                out_ref[:, :] = y * jax.nn.sigmoid(y)

            is_last = (h == N_HOPS - 1)
            if not is_last:
                pl.semaphore_signal(
                    credit_sem, inc=1,
                    device_id=(left,), device_id_type=pl.DeviceIdType.MESH,
                )
            else:
                @pl.when(t < N_BLKS - 1)
                def _():
                    pl.semaphore_signal(
                        credit_sem, inc=1,
                        device_id=(left,), device_id_type=pl.DeviceIdType.MESH,
                    )

    grid = (N_BLKS,)
    return pl.pallas_call(
        body,
        grid=grid,
        in_specs=[
            pl.BlockSpec((4 * M_PER, K_PER), lambda t: (0, 0),
                         memory_space=pltpu.VMEM),
            pl.BlockSpec((K_PER, BLK_N), lambda t: (0, t),
                         memory_space=pltpu.VMEM),
            pl.BlockSpec(memory_space=pltpu.SMEM),
            pl.BlockSpec(memory_space=pltpu.SMEM),
        ],
        out_specs=pl.BlockSpec((M_PER, BLK_N), lambda t: (0, t),
                               memory_space=pltpu.VMEM),
        out_shape=jax.ShapeDtypeStruct((M_PER, N), jnp.float32),
        scratch_shapes=[
            pltpu.VMEM((2, M_PER, BLK_N), jnp.bfloat16),
            pltpu.SemaphoreType.DMA((2,)),
            pltpu.SemaphoreType.DMA((2,)),
            pltpu.SemaphoreType.REGULAR,
        ],
        compiler_params=pltpu.CompilerParams(
            collective_id=0,
            dimension_semantics=("arbitrary",),
        ),
    )(x8, w8, scale_x, scale_w)


# baseline (device time: 662736 ns/iter reference)
import jax
import jax.numpy as jnp
from jax import lax
from jax.experimental import pallas as pl
from jax.experimental.pallas import tpu as pltpu

N_DEV = 4
M_PER = 1024
K_PER = 1024
N = 8192
BLK_N = 2048
N_BLKS = N // BLK_N
N_HOPS = N_DEV - 1
TOTAL_HOPS = N_BLKS * N_HOPS
SUB_N = 1024
N_SUB = BLK_N // SUB_N


def kernel(x, w_mat, scale_x, scale_w):
    x8 = x.astype(jnp.float8_e5m2)
    w8 = w_mat.astype(jnp.float8_e5m2)

    def body(x_ref, w_ref, sx_ref, sw_ref, out_ref,
             comm_ref, send_sems, recv_sems, credit_sem):
        t = pl.program_id(0)
        my = lax.axis_index("i")
        left = lax.rem(my + N_DEV - 1, N_DEV)
        right = lax.rem(my + 1, N_DEV)

        @pl.when(t == 0)
        def _entry_barrier():
            barrier_sem = pltpu.get_barrier_semaphore()
            for nbr in (left, right):
                pl.semaphore_signal(
                    barrier_sem, inc=1,
                    device_id=(nbr,), device_id_type=pl.DeviceIdType.MESH,
                )
            pl.semaphore_wait(barrier_sem, 2)

        def partial_sub(chunk, j):
            xc = x_ref[pl.ds(chunk * M_PER, M_PER), :]
            return lax.dot_general(
                xc, w_ref[:, j * SUB_N:(j + 1) * SUB_N],
                (((1,), (0,)), ((), ())),
                preferred_element_type=jnp.float32,
            )

        send_slot0 = lax.rem(t, 2)
        c0 = lax.rem(my + N_DEV - 1, N_DEV)
        for j in range(N_SUB):
            comm_ref[send_slot0, :, j * SUB_N:(j + 1) * SUB_N] = (
                partial_sub(c0, j).astype(jnp.bfloat16))

        for h in range(N_HOPS):
            send_slot = lax.rem(t + h, 2)
            recv_slot = lax.rem(t + h + 1, 2)

            if h > 0:
                pl.semaphore_wait(credit_sem, 1)
            else:
                @pl.when(t > 0)
                def _():
                    pl.semaphore_wait(credit_sem, 1)

            rdma = pltpu.make_async_remote_copy(
                src_ref=comm_ref.at[send_slot],
                dst_ref=comm_ref.at[recv_slot],
                send_sem=send_sems.at[send_slot],
                recv_sem=recv_sems.at[recv_slot],
                device_id=(right,),
                device_id_type=pl.DeviceIdType.MESH,
            )
            rdma.start()

            c = lax.rem(my + 2 * N_DEV - 2 - h, N_DEV)

            for j in range(N_SUB):
                part = partial_sub(c, j)
                if j == 0:
                    rdma.wait()
                sl = slice(j * SUB_N, (j + 1) * SUB_N)
                if h < N_HOPS - 1:
                    comm_ref[recv_slot, :, sl] = (
                        comm_ref[recv_slot, :, sl].astype(jnp.float32) + part
                    ).astype(jnp.bfloat16)
                else:
                    acc = comm_ref[recv_slot, :, sl].astype(jnp.float32) + part
                    y = acc * (sx_ref[0] * sw_ref[0])
                    out_ref[:, sl] = y * jax.nn.sigmoid(y)

            is_last = (h == N_HOPS - 1)
            if not is_last:
                pl.semaphore_signal(
                    credit_sem, inc=1,
                    device_id=(left,), device_id_type=pl.DeviceIdType.MESH,
                )
            else:
                @pl.when(t < N_BLKS - 1)
                def _():
                    pl.semaphore_signal(
                        credit_sem, inc=1,
                        device_id=(left,), device_id_type=pl.DeviceIdType.MESH,
                    )

    grid = (N_BLKS,)
    return pl.pallas_call(
        body,
        grid=grid,
        in_specs=[
            pl.BlockSpec((4 * M_PER, K_PER), lambda t: (0, 0),
                         memory_space=pltpu.VMEM),
            pl.BlockSpec((K_PER, BLK_N), lambda t: (0, t),
                         memory_space=pltpu.VMEM),
            pl.BlockSpec(memory_space=pltpu.SMEM),
            pl.BlockSpec(memory_space=pltpu.SMEM),
        ],
        out_specs=pl.BlockSpec((M_PER, BLK_N), lambda t: (0, t),
                               memory_space=pltpu.VMEM),
        out_shape=jax.ShapeDtypeStruct((M_PER, N), jnp.float32),
        scratch_shapes=[
            pltpu.VMEM((2, M_PER, BLK_N), jnp.bfloat16),
            pltpu.SemaphoreType.DMA((2,)),
            pltpu.SemaphoreType.DMA((2,)),
            pltpu.SemaphoreType.REGULAR,
        ],
        compiler_params=pltpu.CompilerParams(
            collective_id=0,
            dimension_semantics=("arbitrary",),
            vmem_limit_bytes=60 * 1024 * 1024,
        ),
    )(x8, w8, scale_x, scale_w)


# device time: 375754 ns/iter; 1.7637x vs baseline; 1.7637x over previous
import jax
import jax.numpy as jnp
from jax import lax
from jax.experimental import pallas as pl
from jax.experimental.pallas import tpu as pltpu

N_DEV = 4
M_PER = 1024
K_PER = 1024
N = 8192
BLK_N = 2048
N_BLKS = N // BLK_N
N_HOPS = N_DEV - 1
HALF = BLK_N // 2


def kernel(x, w_mat, scale_x, scale_w):
    x8 = x.astype(jnp.float8_e5m2)
    w8 = w_mat.astype(jnp.float8_e5m2)

    def body(x_ref, w_ref, sx_ref, sw_ref, out_ref,
             comm_ref, send_sems, recv_sems, credit_sems):
        t = pl.program_id(0)
        my = lax.axis_index("i")
        left = lax.rem(my + N_DEV - 1, N_DEV)
        right = lax.rem(my + 1, N_DEV)
        dirs = ((right, left), (left, right))

        @pl.when(t == 0)
        def _entry_barrier():
            barrier_sem = pltpu.get_barrier_semaphore()
            for nbr in (left, right):
                pl.semaphore_signal(
                    barrier_sem, inc=1,
                    device_id=(nbr,), device_id_type=pl.DeviceIdType.MESH,
                )
            pl.semaphore_wait(barrier_sem, 2)

        def partial_for(chunk, d):
            xc = x_ref[pl.ds(chunk * M_PER, M_PER), :]
            return lax.dot_general(
                xc, w_ref[:, d * HALF:(d + 1) * HALF],
                (((1,), (0,)), ((), ())),
                preferred_element_type=jnp.float32,
            )

        send_slot0 = lax.rem(t, 2)
        for d in range(2):
            c0 = lax.rem(my + N_DEV + (1 if d else -1), N_DEV)
            comm_ref[d, send_slot0] = partial_for(c0, d).astype(jnp.bfloat16)

        for h in range(N_HOPS):
            send_slot = lax.rem(t + h, 2)
            recv_slot = lax.rem(t + h + 1, 2)

            for d in range(2):
                if h > 0:
                    pl.semaphore_wait(credit_sems.at[d], 1)
                else:
                    @pl.when(t > 0)
                    def _(d=d):
                        pl.semaphore_wait(credit_sems.at[d], 1)

            rdmas = []
            for d, (out_nbr, _) in enumerate(dirs):
                rdma = pltpu.make_async_remote_copy(
                    src_ref=comm_ref.at[d, send_slot],
                    dst_ref=comm_ref.at[d, recv_slot],
                    send_sem=send_sems.at[d, send_slot],
                    recv_sem=recv_sems.at[d, recv_slot],
                    device_id=(out_nbr,),
                    device_id_type=pl.DeviceIdType.MESH,
                )
                rdma.start()
                rdmas.append(rdma)

            parts = [
                partial_for(lax.rem(my + 2 * N_DEV - 2 - h, N_DEV), 0),
                partial_for(lax.rem(my + 2 + h, N_DEV), 1),
            ]

            for d in range(2):
                rdmas[d].wait()
                if h < N_HOPS - 1:
                    comm_ref[d, recv_slot] = (
                        comm_ref[d, recv_slot].astype(jnp.float32) + parts[d]
                    ).astype(jnp.bfloat16)
                else:
                    acc = comm_ref[d, recv_slot].astype(jnp.float32) + parts[d]
                    y = acc * (sx_ref[0] * sw_ref[0])
                    out_ref[:, d * HALF:(d + 1) * HALF] = (
                        y * jax.nn.sigmoid(y))

            for d, (_, upstream) in enumerate(dirs):
                if h < N_HOPS - 1:
                    pl.semaphore_signal(
                        credit_sems.at[d], inc=1,
                        device_id=(upstream,),
                        device_id_type=pl.DeviceIdType.MESH,
                    )
                else:
                    @pl.when(t < N_BLKS - 1)
                    def _(d=d, upstream=upstream):
                        pl.semaphore_signal(
                            credit_sems.at[d], inc=1,
                            device_id=(upstream,),
                            device_id_type=pl.DeviceIdType.MESH,
                        )

    grid = (N_BLKS,)
    return pl.pallas_call(
        body,
        grid=grid,
        in_specs=[
            pl.BlockSpec((N_DEV * M_PER, K_PER), lambda t: (0, 0),
                         memory_space=pltpu.VMEM),
            pl.BlockSpec((K_PER, BLK_N), lambda t: (0, t),
                         memory_space=pltpu.VMEM),
            pl.BlockSpec(memory_space=pltpu.SMEM),
            pl.BlockSpec(memory_space=pltpu.SMEM),
        ],
        out_specs=pl.BlockSpec((M_PER, BLK_N), lambda t: (0, t),
                               memory_space=pltpu.VMEM),
        out_shape=jax.ShapeDtypeStruct((M_PER, N), jnp.float32),
        scratch_shapes=[
            pltpu.VMEM((2, 2, M_PER, HALF), jnp.bfloat16),
            pltpu.SemaphoreType.DMA((2, 2)),
            pltpu.SemaphoreType.DMA((2, 2)),
            pltpu.SemaphoreType.REGULAR((2,)),
        ],
        compiler_params=pltpu.CompilerParams(
            collective_id=0,
            dimension_semantics=("arbitrary",),
            vmem_limit_bytes=60 * 1024 * 1024,
        ),
    )(x8, w8, scale_x, scale_w)


# device time: 372498 ns/iter; 1.7792x vs baseline; 1.0087x over previous
import jax
import jax.numpy as jnp
from jax import lax
from jax.experimental import pallas as pl
from jax.experimental.pallas import tpu as pltpu

N_DEV = 4
M_PER = 1024
K_PER = 1024
N = 8192
BLK_N = 2048
N_BLKS = N // BLK_N
N_HOPS = N_DEV - 1
HALF = BLK_N // 2


def kernel(x, w_mat, scale_x, scale_w):
    x8 = x.astype(jnp.float8_e5m2)
    w8 = w_mat.astype(jnp.float8_e5m2)

    def body(x_ref, w_ref, sx_ref, sw_ref, out_ref,
             comm_ref, send_sems, recv_sems, credit_sems):
        t = pl.program_id(0)
        my = lax.axis_index("i")
        left = lax.rem(my + N_DEV - 1, N_DEV)
        right = lax.rem(my + 1, N_DEV)
        dirs = ((right, left), (left, right))

        @pl.when(t == 0)
        def _entry_barrier():
            barrier_sem = pltpu.get_barrier_semaphore()
            for nbr in (left, right):
                pl.semaphore_signal(
                    barrier_sem, inc=1,
                    device_id=(nbr,), device_id_type=pl.DeviceIdType.MESH,
                )
            pl.semaphore_wait(barrier_sem, 2)

        def partial_for(chunk, d):
            xc = x_ref[pl.ds(chunk * M_PER, M_PER), :]
            return lax.dot_general(
                xc, w_ref[:, d * HALF:(d + 1) * HALF],
                (((1,), (0,)), ((), ())),
                preferred_element_type=jnp.float32,
            )

        send_slot0 = lax.rem(t, 2)
        for d in range(2):
            c0 = lax.rem(my + N_DEV + (1 if d else -1), N_DEV)
            comm_ref[d, send_slot0] = partial_for(c0, d).astype(jnp.bfloat16)

        for h in range(N_HOPS):
            send_slot = lax.rem(t + h, 2)
            recv_slot = lax.rem(t + h + 1, 2)

            for d in range(2):
                if h > 0:
                    pl.semaphore_wait(credit_sems.at[d], 1)
                else:
                    @pl.when(t > 0)
                    def _(d=d):
                        pl.semaphore_wait(credit_sems.at[d], 1)

            rdmas = []
            for d, (out_nbr, _) in enumerate(dirs):
                rdma = pltpu.make_async_remote_copy(
                    src_ref=comm_ref.at[d, send_slot],
                    dst_ref=comm_ref.at[d, recv_slot],
                    send_sem=send_sems.at[d, send_slot],
                    recv_sem=recv_sems.at[d, recv_slot],
                    device_id=(out_nbr,),
                    device_id_type=pl.DeviceIdType.MESH,
                )
                rdma.start()
                rdmas.append(rdma)

            parts = [
                partial_for(lax.rem(my + 2 * N_DEV - 2 - h, N_DEV), 0),
                partial_for(lax.rem(my + 2 + h, N_DEV), 1),
            ]

            for d, (_, upstream) in enumerate(dirs):
                rdmas[d].wait()
                if h < N_HOPS - 1:
                    pl.semaphore_signal(
                        credit_sems.at[d], inc=1,
                        device_id=(upstream,),
                        device_id_type=pl.DeviceIdType.MESH,
                    )
                else:
                    @pl.when(t < N_BLKS - 1)
                    def _(d=d, upstream=upstream):
                        pl.semaphore_signal(
                            credit_sems.at[d], inc=1,
                            device_id=(upstream,),
                            device_id_type=pl.DeviceIdType.MESH,
                        )
                if h < N_HOPS - 1:
                    comm_ref[d, recv_slot] = (
                        comm_ref[d, recv_slot][...]
                        + parts[d].astype(jnp.bfloat16))
                else:
                    acc = comm_ref[d, recv_slot].astype(jnp.float32) + parts[d]
                    y = acc * (sx_ref[0] * sw_ref[0])
                    out_ref[:, d * HALF:(d + 1) * HALF] = (
                        y * jax.nn.sigmoid(y))

    grid = (N_BLKS,)
    return pl.pallas_call(
        body,
        grid=grid,
        in_specs=[
            pl.BlockSpec((N_DEV * M_PER, K_PER), lambda t: (0, 0),
                         memory_space=pltpu.VMEM),
            pl.BlockSpec((K_PER, BLK_N), lambda t: (0, t),
                         memory_space=pltpu.VMEM),
            pl.BlockSpec(memory_space=pltpu.SMEM),
            pl.BlockSpec(memory_space=pltpu.SMEM),
        ],
        out_specs=pl.BlockSpec((M_PER, BLK_N), lambda t: (0, t),
                               memory_space=pltpu.VMEM),
        out_shape=jax.ShapeDtypeStruct((M_PER, N), jnp.float32),
        scratch_shapes=[
            pltpu.VMEM((2, 2, M_PER, HALF), jnp.bfloat16),
            pltpu.SemaphoreType.DMA((2, 2)),
            pltpu.SemaphoreType.DMA((2, 2)),
            pltpu.SemaphoreType.REGULAR((2,)),
        ],
        compiler_params=pltpu.CompilerParams(
            collective_id=0,
            dimension_semantics=("arbitrary",),
            vmem_limit_bytes=60 * 1024 * 1024,
        ),
    )(x8, w8, scale_x, scale_w)


# device time: 330679 ns/iter; 2.0042x vs baseline; 1.1265x over previous
import jax
import jax.numpy as jnp
from jax import lax
from jax.experimental import pallas as pl
from jax.experimental.pallas import tpu as pltpu

N_DEV = 4
M_PER = 1024
K_PER = 1024
N = 8192
BLK_N = 4096
N_BLKS = N // BLK_N
N_HOPS = N_DEV - 1
HALF = BLK_N // 2
SUB = 1024
N_SUB = HALF // SUB


def kernel(x, w_mat, scale_x, scale_w):
    x8 = x.astype(jnp.float8_e5m2)
    w8 = w_mat.astype(jnp.float8_e5m2)

    def body(x_ref, w_ref, sx_ref, sw_ref, out_ref,
             comm_ref, send_sems, recv_sems, credit_sems):
        t = pl.program_id(0)
        my = lax.axis_index("i")
        left = lax.rem(my + N_DEV - 1, N_DEV)
        right = lax.rem(my + 1, N_DEV)
        dirs = ((right, left), (left, right))
        scale = sx_ref[0] * sw_ref[0]

        @pl.when(t == 0)
        def _entry_barrier():
            barrier_sem = pltpu.get_barrier_semaphore()
            for nbr in (left, right):
                pl.semaphore_signal(
                    barrier_sem, inc=1,
                    device_id=(nbr,), device_id_type=pl.DeviceIdType.MESH,
                )
            pl.semaphore_wait(barrier_sem, 2)

        def partial(chunk, d, u):
            xc = x_ref[pl.ds(chunk * M_PER, M_PER), :]
            wc = w_ref[:, pl.ds(t * BLK_N + d * HALF + u * SUB, SUB)]
            return lax.dot_general(
                xc, wc, (((1,), (0,)), ((), ())),
                preferred_element_type=jnp.float32,
            )

        def mk_rdma(d, u, s_slot, r_slot):
            return pltpu.make_async_remote_copy(
                src_ref=comm_ref.at[d, s_slot, u],
                dst_ref=comm_ref.at[d, r_slot, u],
                send_sem=send_sems.at[d, s_slot, u],
                recv_sem=recv_sems.at[d, r_slot, u],
                device_id=(dirs[d][0],),
                device_id_type=pl.DeviceIdType.MESH,
            )

        def credit_wait(d, u):
            pl.semaphore_wait(credit_sems.at[d, u], 1)

        def credit_signal(d, u):
            pl.semaphore_signal(
                credit_sems.at[d, u], inc=1,
                device_id=(dirs[d][1],),
                device_id_type=pl.DeviceIdType.MESH,
            )

        slot0 = lax.rem(t, 2)
        slot1 = lax.rem(t + 1, 2)
        inflight = {}
        for u in range(N_SUB):
            for d in range(2):
                c0 = lax.rem(my + N_DEV + (1 if d else -1), N_DEV)
                comm_ref[d, slot0, u] = partial(c0, d, u).astype(jnp.bfloat16)

                @pl.when(t > 0)
                def _(d=d, u=u):
                    credit_wait(d, u)
                r = mk_rdma(d, u, slot0, slot1)
                r.start()
                inflight[(d, u)] = r

        for h in range(N_HOPS):
            r_slot = lax.rem(t + h + 1, 2)
            n_slot = lax.rem(t + h, 2)
            last = h == N_HOPS - 1
            for u in range(N_SUB):
                for d in range(2):
                    c = lax.rem(my + (2 + h if d else 2 * N_DEV - 2 - h),
                                N_DEV)
                    part = partial(c, d, u)
                    inflight[(d, u)].wait()
                    if not last:
                        credit_signal(d, u)
                    else:
                        @pl.when(t < N_BLKS - 1)
                        def _(d=d, u=u):
                            credit_signal(d, u)
                    if not last:
                        comm_ref[d, r_slot, u] = (
                            comm_ref[d, r_slot, u][...]
                            + part.astype(jnp.bfloat16))
                        credit_wait(d, u)
                        r = mk_rdma(d, u, r_slot, n_slot)
                        r.start()
                        inflight[(d, u)] = r
                    else:
                        acc = comm_ref[d, r_slot, u].astype(jnp.float32) + part
                        y = acc * scale
                        col = d * HALF + u * SUB
                        out_ref[:, col:col + SUB] = (
                            y * jax.nn.sigmoid(y)).astype(jnp.bfloat16)

    grid = (N_BLKS,)
    return pl.pallas_call(
        body,
        grid=grid,
        in_specs=[
            pl.BlockSpec((N_DEV * M_PER, K_PER), lambda t: (0, 0),
                         memory_space=pltpu.VMEM),
            pl.BlockSpec((K_PER, N), lambda t: (0, 0),
                         memory_space=pltpu.VMEM),
            pl.BlockSpec(memory_space=pltpu.SMEM),
            pl.BlockSpec(memory_space=pltpu.SMEM),
        ],
        out_specs=pl.BlockSpec((M_PER, BLK_N), lambda t: (0, t),
                               memory_space=pltpu.VMEM),
        out_shape=jax.ShapeDtypeStruct((M_PER, N), jnp.bfloat16),
        scratch_shapes=[
            pltpu.VMEM((2, 2, N_SUB, M_PER, SUB), jnp.bfloat16),
            pltpu.SemaphoreType.DMA((2, 2, N_SUB)),
            pltpu.SemaphoreType.DMA((2, 2, N_SUB)),
            pltpu.SemaphoreType.REGULAR((2, N_SUB)),
        ],
        compiler_params=pltpu.CompilerParams(
            collective_id=0,
            dimension_semantics=("arbitrary",),
            vmem_limit_bytes=60 * 1024 * 1024,
        ),
    )(x8, w8, scale_x, scale_w)


# device time: 320845 ns/iter; 2.0656x vs baseline; 1.0307x over previous
import jax
import jax.numpy as jnp
from jax import lax
from jax.experimental import pallas as pl
from jax.experimental.pallas import tpu as pltpu

N_DEV = 4
M_PER = 1024
K_PER = 1024
N = 8192
N_HOPS = N_DEV - 1
HALF = N // 2
SUB = 512
N_SUB = HALF // SUB


def kernel(x, w_mat, scale_x, scale_w):
    x8 = x.astype(jnp.float8_e5m2)
    w8 = w_mat.astype(jnp.float8_e5m2)

    def body(x_ref, w_ref, sx_ref, sw_ref, out_ref,
             comm_ref, stage_ref, send_sems, recv_sems, credit_sems,
             out_sems):
        my = lax.axis_index("i")
        left = lax.rem(my + N_DEV - 1, N_DEV)
        right = lax.rem(my + 1, N_DEV)
        dirs = ((right, left), (left, right))
        scale = sx_ref[0] * sw_ref[0]

        barrier_sem = pltpu.get_barrier_semaphore()
        for nbr in (left, right):
            pl.semaphore_signal(
                barrier_sem, inc=1,
                device_id=(nbr,), device_id_type=pl.DeviceIdType.MESH,
            )
        pl.semaphore_wait(barrier_sem, 2)

        def partial(chunk, d, u):
            xc = x_ref[pl.ds(chunk * M_PER, M_PER), :]
            wc = w_ref[:, d * HALF + u * SUB:d * HALF + (u + 1) * SUB]
            return lax.dot_general(
                xc, wc, (((1,), (0,)), ((), ())),
                preferred_element_type=jnp.float32,
            )

        def mk_rdma(d, u, s_slot, r_slot):
            return pltpu.make_async_remote_copy(
                src_ref=comm_ref.at[d, s_slot, u],
                dst_ref=comm_ref.at[d, r_slot, u],
                send_sem=send_sems.at[d, s_slot, u],
                recv_sem=recv_sems.at[d, r_slot, u],
                device_id=(dirs[d][0],),
                device_id_type=pl.DeviceIdType.MESH,
            )

        inflight = {}
        for u in range(N_SUB):
            for d in range(2):
                c0 = lax.rem(my + N_DEV + (1 if d else -1), N_DEV)
                comm_ref[d, 0, u] = partial(c0, d, u).astype(jnp.bfloat16)
                r = mk_rdma(d, u, 0, 1)
                r.start()
                inflight[(d, u)] = r

        n_tile = 0
        out_cps = {}
        for h in range(N_HOPS):
            s_slot = h % 2
            r_slot = (h + 1) % 2
            last = h == N_HOPS - 1
            for u in range(N_SUB):
                for d in range(2):
                    c = lax.rem(my + (2 + h if d else 2 * N_DEV - 2 - h),
                                N_DEV)
                    part = partial(c, d, u)
                    inflight[(d, u)].wait()
                    if not last:
                        pl.semaphore_signal(
                            credit_sems.at[d, u], inc=1,
                            device_id=(dirs[d][1],),
                            device_id_type=pl.DeviceIdType.MESH,
                        )
                        comm_ref[d, r_slot, u] = (
                            comm_ref[d, r_slot, u][...]
                            + part.astype(jnp.bfloat16))
                        pl.semaphore_wait(credit_sems.at[d, u], 1)
                        r = mk_rdma(d, u, r_slot, s_slot)
                        r.start()
                        inflight[(d, u)] = r
                    else:
                        buf = n_tile % 2
                        if n_tile >= 2:
                            out_cps[buf].wait()
                        acc = (comm_ref[d, r_slot, u].astype(jnp.float32)
                               + part)
                        y = acc * scale
                        stage_ref[buf] = (y * jax.nn.sigmoid(y)
                                          ).astype(jnp.bfloat16)
                        col = d * HALF + u * SUB
                        cp = pltpu.make_async_copy(
                            stage_ref.at[buf],
                            out_ref.at[:, pl.ds(col, SUB)],
                            out_sems.at[buf],
                        )
                        cp.start()
                        out_cps[buf] = cp
                        n_tile += 1

        out_cps[0].wait()
        out_cps[1].wait()

    return pl.pallas_call(
        body,
        in_specs=[
            pl.BlockSpec(memory_space=pltpu.VMEM),
            pl.BlockSpec(memory_space=pltpu.VMEM),
            pl.BlockSpec(memory_space=pltpu.SMEM),
            pl.BlockSpec(memory_space=pltpu.SMEM),
        ],
        out_specs=pl.BlockSpec(memory_space=pl.ANY),
        out_shape=jax.ShapeDtypeStruct((M_PER, N), jnp.bfloat16),
        scratch_shapes=[
            pltpu.VMEM((2, 2, N_SUB, M_PER, SUB), jnp.bfloat16),
            pltpu.VMEM((2, M_PER, SUB), jnp.bfloat16),
            pltpu.SemaphoreType.DMA((2, 2, N_SUB)),
            pltpu.SemaphoreType.DMA((2, 2, N_SUB)),
            pltpu.SemaphoreType.REGULAR((2, N_SUB)),
            pltpu.SemaphoreType.DMA((2,)),
        ],
        compiler_params=pltpu.CompilerParams(
            collective_id=0,
            vmem_limit_bytes=62 * 1024 * 1024,
        ),
    )(x8, w8, scale_x, scale_w)


# device time: 305662 ns/iter; 2.1682x vs baseline; 1.0497x over previous
import jax
import jax.numpy as jnp
from jax import lax
from jax.experimental import pallas as pl
from jax.experimental.pallas import tpu as pltpu

N_DEV = 4
M_PER = 1024
K_PER = 1024
N = 8192
N_HOPS = N_DEV - 1
HALF = N // 2
SUB = 512
N_SUB = HALF // SUB


def kernel(x, w_mat, scale_x, scale_w):
    x8 = x.astype(jnp.float8_e5m2)

    def body(x_ref, w_ref, sx_ref, sw_ref, out_ref,
             comm_ref, w8_ref, wstg_ref, stage_ref,
             send_sems, recv_sems, credit_sems, out_sems, wstg_sems):
        my = lax.axis_index("i")
        left = lax.rem(my + N_DEV - 1, N_DEV)
        right = lax.rem(my + 1, N_DEV)
        dirs = ((right, left), (left, right))
        scale = sx_ref[0] * sw_ref[0]

        barrier_sem = pltpu.get_barrier_semaphore()
        for nbr in (left, right):
            pl.semaphore_signal(
                barrier_sem, inc=1,
                device_id=(nbr,), device_id_type=pl.DeviceIdType.MESH,
            )
        pl.semaphore_wait(barrier_sem, 2)

        def partial(chunk, d, u):
            xc = x_ref[pl.ds(chunk * M_PER, M_PER), :]
            wc = w8_ref[:, d * HALF + u * SUB:d * HALF + (u + 1) * SUB]
            return lax.dot_general(
                xc, wc, (((1,), (0,)), ((), ())),
                preferred_element_type=jnp.float32,
            )

        def w_stripe_dma(s, buf):
            u, d = divmod(s, 2)
            col = d * HALF + u * SUB
            return pltpu.make_async_copy(
                w_ref.at[:, pl.ds(col, SUB)],
                wstg_ref.at[buf],
                wstg_sems.at[buf],
            )

        def mk_rdma(d, u, s_slot, r_slot):
            return pltpu.make_async_remote_copy(
                src_ref=comm_ref.at[d, s_slot, u],
                dst_ref=comm_ref.at[d, r_slot, u],
                send_sem=send_sems.at[d, s_slot, u],
                recv_sem=recv_sems.at[d, r_slot, u],
                device_id=(dirs[d][0],),
                device_id_type=pl.DeviceIdType.MESH,
            )

        w_stripe_dma(0, 0).start()
        inflight = {}
        for u in range(N_SUB):
            for d in range(2):
                s = u * 2 + d
                buf = s % 2
                w_stripe_dma(s, buf).wait()
                if s + 1 < 2 * N_SUB:
                    w_stripe_dma(s + 1, (s + 1) % 2).start()
                col = d * HALF + u * SUB
                w8_ref[:, col:col + SUB] = (
                    wstg_ref[buf].astype(jnp.float8_e5m2))
                c0 = lax.rem(my + N_DEV + (1 if d else -1), N_DEV)
                comm_ref[d, 0, u] = partial(c0, d, u).astype(jnp.bfloat16)
                r = mk_rdma(d, u, 0, 1)
                r.start()
                inflight[(d, u)] = r

        n_tile = 0
        out_cps = {}
        for h in range(N_HOPS):
            s_slot = h % 2
            r_slot = (h + 1) % 2
            last = h == N_HOPS - 1
            for u in range(N_SUB):
                for d in range(2):
                    c = lax.rem(my + (2 + h if d else 2 * N_DEV - 2 - h),
                                N_DEV)
                    part = partial(c, d, u)
                    inflight[(d, u)].wait()
                    if not last:
                        pl.semaphore_signal(
                            credit_sems.at[d, u], inc=1,
                            device_id=(dirs[d][1],),
                            device_id_type=pl.DeviceIdType.MESH,
                        )
                        comm_ref[d, r_slot, u] = (
                            comm_ref[d, r_slot, u][...]
                            + part.astype(jnp.bfloat16))
                        pl.semaphore_wait(credit_sems.at[d, u], 1)
                        r = mk_rdma(d, u, r_slot, s_slot)
                        r.start()
                        inflight[(d, u)] = r
                    else:
                        buf = n_tile % 2
                        if n_tile >= 2:
                            out_cps[buf].wait()
                        acc = (comm_ref[d, r_slot, u].astype(jnp.float32)
                               + part)
                        y = acc * scale
                        stage_ref[buf] = (y * jax.nn.sigmoid(y)
                                          ).astype(jnp.bfloat16)
                        col = d * HALF + u * SUB
                        cp = pltpu.make_async_copy(
                            stage_ref.at[buf],
                            out_ref.at[:, pl.ds(col, SUB)],
                            out_sems.at[buf],
                        )
                        cp.start()
                        out_cps[buf] = cp
                        n_tile += 1

        out_cps[0].wait()
        out_cps[1].wait()

    return pl.pallas_call(
        body,
        in_specs=[
            pl.BlockSpec(memory_space=pltpu.VMEM),
            pl.BlockSpec(memory_space=pl.ANY),
            pl.BlockSpec(memory_space=pltpu.SMEM),
            pl.BlockSpec(memory_space=pltpu.SMEM),
        ],
        out_specs=pl.BlockSpec(memory_space=pl.ANY),
        out_shape=jax.ShapeDtypeStruct((M_PER, N), jnp.bfloat16),
        scratch_shapes=[
            pltpu.VMEM((2, 2, N_SUB, M_PER, SUB), jnp.bfloat16),
            pltpu.VMEM((K_PER, N), jnp.float8_e5m2),
            pltpu.VMEM((2, K_PER, SUB), jnp.float32),
            pltpu.VMEM((2, M_PER, SUB), jnp.bfloat16),
            pltpu.SemaphoreType.DMA((2, 2, N_SUB)),
            pltpu.SemaphoreType.DMA((2, 2, N_SUB)),
            pltpu.SemaphoreType.REGULAR((2, N_SUB)),
            pltpu.SemaphoreType.DMA((2,)),
            pltpu.SemaphoreType.DMA((2,)),
        ],
        compiler_params=pltpu.CompilerParams(
            collective_id=0,
            vmem_limit_bytes=62 * 1024 * 1024,
        ),
    )(x8, w_mat, scale_x, scale_w)


# device time: 298306 ns/iter; 2.2217x vs baseline; 1.0247x over previous
import jax
import jax.numpy as jnp
from jax import lax
from jax.experimental import pallas as pl
from jax.experimental.pallas import tpu as pltpu

N_DEV = 4
M_PER = 1024
K_PER = 1024
N = 8192
N_HOPS = N_DEV - 1
HALF = N // 2
SUB = 512
N_SUB = HALF // SUB


def kernel(x, w_mat, scale_x, scale_w):

    def body(x_ref, w_ref, sx_ref, sw_ref, out_ref,
             comm_ref, x8_ref, xstg_ref, w8_ref, wstg_ref, stage_ref,
             send_sems, recv_sems, credit_sems, out_sems, wstg_sems,
             xstg_sems):
        my = lax.axis_index("i")
        left = lax.rem(my + N_DEV - 1, N_DEV)
        right = lax.rem(my + 1, N_DEV)
        dirs = ((right, left), (left, right))
        scale = sx_ref[0] * sw_ref[0]

        barrier_sem = pltpu.get_barrier_semaphore()
        for nbr in (left, right):
            pl.semaphore_signal(
                barrier_sem, inc=1,
                device_id=(nbr,), device_id_type=pl.DeviceIdType.MESH,
            )
        pl.semaphore_wait(barrier_sem, 2)

        def x_chunk_convert(chunk):
            cp = pltpu.make_async_copy(
                x_ref.at[pl.ds(chunk * M_PER, M_PER), :],
                xstg_ref, xstg_sems,
            )
            cp.start()
            cp.wait()
            x8_ref[pl.ds(chunk * M_PER, M_PER), :] = (
                xstg_ref[...].astype(jnp.float8_e5m2))

        def partial(chunk, d, u):
            xc = x8_ref[pl.ds(chunk * M_PER, M_PER), :]
            wc = w8_ref[:, d * HALF + u * SUB:d * HALF + (u + 1) * SUB]
            return lax.dot_general(
                xc, wc, (((1,), (0,)), ((), ())),
                preferred_element_type=jnp.float32,
            )

        def w_stripe_dma(s, buf):
            u, d = divmod(s, 2)
            col = d * HALF + u * SUB
            return pltpu.make_async_copy(
                w_ref.at[:, pl.ds(col, SUB)],
                wstg_ref.at[buf],
                wstg_sems.at[buf],
            )

        def mk_rdma(d, u, s_slot, r_slot):
            return pltpu.make_async_remote_copy(
                src_ref=comm_ref.at[d, s_slot, u],
                dst_ref=comm_ref.at[d, r_slot, u],
                send_sem=send_sems.at[d, s_slot, u],
                recv_sem=recv_sems.at[d, r_slot, u],
                device_id=(dirs[d][0],),
                device_id_type=pl.DeviceIdType.MESH,
            )

        w_stripe_dma(0, 0).start()
        x_chunk_convert(lax.rem(my + N_DEV - 1, N_DEV))
        x_chunk_convert(lax.rem(my + 1, N_DEV))
        inflight = {}
        for u in range(N_SUB):
            for d in range(2):
                s = u * 2 + d
                buf = s % 2
                w_stripe_dma(s, buf).wait()
                if s + 1 < 2 * N_SUB:
                    w_stripe_dma(s + 1, (s + 1) % 2).start()
                col = d * HALF + u * SUB
                w8_ref[:, col:col + SUB] = (
                    wstg_ref[buf].astype(jnp.float8_e5m2))
                c0 = lax.rem(my + N_DEV + (1 if d else -1), N_DEV)
                comm_ref[d, 0, u] = partial(c0, d, u).astype(jnp.bfloat16)
                r = mk_rdma(d, u, 0, 1)
                r.start()
                inflight[(d, u)] = r

        x_chunk_convert(lax.rem(my + 2, N_DEV))

        n_tile = 0
        out_cps = {}
        for h in range(N_HOPS):
            s_slot = h % 2
            r_slot = (h + 1) % 2
            last = h == N_HOPS - 1
            if h == 1:
                x_chunk_convert(my)
            for u in range(N_SUB):
                for d in range(2):
                    c = lax.rem(my + (2 + h if d else 2 * N_DEV - 2 - h),
                                N_DEV)
                    part = partial(c, d, u)
                    inflight[(d, u)].wait()
                    if not last:
                        pl.semaphore_signal(
                            credit_sems.at[d, u], inc=1,
                            device_id=(dirs[d][1],),
                            device_id_type=pl.DeviceIdType.MESH,
                        )
                        comm_ref[d, r_slot, u] = (
                            comm_ref[d, r_slot, u][...]
                            + part.astype(jnp.bfloat16))
                        pl.semaphore_wait(credit_sems.at[d, u], 1)
                        r = mk_rdma(d, u, r_slot, s_slot)
                        r.start()
                        inflight[(d, u)] = r
                    else:
                        buf = n_tile % 2
                        if n_tile >= 2:
                            out_cps[buf].wait()
                        acc = (comm_ref[d, r_slot, u].astype(jnp.float32)
                               + part)
                        y = acc * scale
                        stage_ref[buf] = (y * jax.nn.sigmoid(y)
                                          ).astype(jnp.bfloat16)
                        col = d * HALF + u * SUB
                        cp = pltpu.make_async_copy(
                            stage_ref.at[buf],
                            out_ref.at[:, pl.ds(col, SUB)],
                            out_sems.at[buf],
                        )
                        cp.start()
                        out_cps[buf] = cp
                        n_tile += 1

        out_cps[0].wait()
        out_cps[1].wait()

    return pl.pallas_call(
        body,
        in_specs=[
            pl.BlockSpec(memory_space=pl.ANY),
            pl.BlockSpec(memory_space=pl.ANY),
            pl.BlockSpec(memory_space=pltpu.SMEM),
            pl.BlockSpec(memory_space=pltpu.SMEM),
        ],
        out_specs=pl.BlockSpec(memory_space=pl.ANY),
        out_shape=jax.ShapeDtypeStruct((M_PER, N), jnp.bfloat16),
        scratch_shapes=[
            pltpu.VMEM((2, 2, N_SUB, M_PER, SUB), jnp.bfloat16),
            pltpu.VMEM((N_DEV * M_PER, K_PER), jnp.float8_e5m2),
            pltpu.VMEM((M_PER, K_PER), jnp.float32),
            pltpu.VMEM((K_PER, N), jnp.float8_e5m2),
            pltpu.VMEM((2, K_PER, SUB), jnp.float32),
            pltpu.VMEM((2, M_PER, SUB), jnp.bfloat16),
            pltpu.SemaphoreType.DMA((2, 2, N_SUB)),
            pltpu.SemaphoreType.DMA((2, 2, N_SUB)),
            pltpu.SemaphoreType.REGULAR((2, N_SUB)),
            pltpu.SemaphoreType.DMA((2,)),
            pltpu.SemaphoreType.DMA((2,)),
            pltpu.SemaphoreType.DMA(()),
        ],
        compiler_params=pltpu.CompilerParams(
            collective_id=0,
            vmem_limit_bytes=62 * 1024 * 1024,
        ),
    )(x, w_mat, scale_x, scale_w)
